# baseline (speedup 1.0000x reference)
import functools

import jax
import jax.numpy as jnp
from jax import lax
from jax.experimental import pallas as pl
from jax.experimental.pallas import tpu as pltpu, tpu_sc as plsc

_ALPHA = 0.7
_BETA = 0.3
_SMOOTH = 1.0

_INFO = plsc.get_sparse_core_info()
_NC, _NS, _L = _INFO.num_cores, _INFO.num_subcores, _INFO.num_lanes
_NW = _NC * _NS
_CH = 8192


def _make_sc_sums(B, C, V):
    per_w = V // _NW
    n_ch = per_w // _CH
    mesh = plsc.VectorSubcoreMesh(core_axis_name="c", subcore_axis_name="s")

    @functools.partial(
        pl.kernel,
        mesh=mesh,
        out_type=jax.ShapeDtypeStruct((_NW * B * 3 * _L,), jnp.float32),
        scratch_types=[
            pltpu.VMEM((_CH,), jnp.float32),
            pltpu.VMEM((_CH,), jnp.int32),
            pltpu.VMEM((B * 3 * _L,), jnp.float32),
        ],
    )
    def sc_sums(x_hbm, t_hbm, out_hbm, xv, tv, outv):
        wid = lax.axis_index("s") * _NC + lax.axis_index("c")
        base = wid * per_w
        for b in range(B):
            S = jnp.zeros((_L,), jnp.float32)
            T = jnp.zeros((_L,), jnp.float32)
            N = jnp.zeros((_L,), jnp.float32)
            for ci in range(n_ch):
                off = base + ci * _CH
                pltpu.sync_copy(x_hbm.at[b * C + (C - 1), pl.ds(off, _CH)], xv)
                pltpu.sync_copy(t_hbm.at[b, pl.ds(off, _CH)], tv)

                def inner(i, carry):
                    s, t, n = carry
                    xs = xv[pl.ds(i * _L, _L)]
                    ts = tv[pl.ds(i * _L, _L)]
                    m = ts == (C - 1)
                    s = s + xs
                    t = t + jnp.where(m, xs, 0.0)
                    n = n + jnp.where(m, 1.0, 0.0)
                    return s, t, n

                S, T, N = lax.fori_loop(0, _CH // _L, inner, (S, T, N),
                                        unroll=8)
            outv[pl.ds((b * 3 + 0) * _L, _L)] = S
            outv[pl.ds((b * 3 + 1) * _L, _L)] = T
            outv[pl.ds((b * 3 + 2) * _L, _L)] = N
        pltpu.sync_copy(outv, out_hbm.at[pl.ds(wid * B * 3 * _L, B * 3 * _L)])

    return sc_sums


def kernel(inputs, targets):
    B, C, D, H, W = inputs.shape
    V = D * H * W
    xf = inputs.reshape(B * C, V)
    tf = targets.reshape(B, V)
    part = _make_sc_sums(B, C, V)(xf, tf)
    sums = part.reshape(_NW, B, 3, _L).sum(axis=(0, 3))
    S, T, N = sums[:, 0], sums[:, 1], sums[:, 2]
    tversky = (T + _SMOOTH) / (T + _ALPHA * (N - T) + _BETA * (S - T) + _SMOOTH)
    return -tversky.mean()


# speedup vs baseline: 1.7396x; 1.7396x over previous
import functools

import jax
import jax.numpy as jnp
from jax import lax
from jax.experimental import pallas as pl
from jax.experimental.pallas import tpu as pltpu, tpu_sc as plsc

_ALPHA = 0.7
_BETA = 0.3
_SMOOTH = 1.0

_INFO = plsc.get_sparse_core_info()
_NC, _NS, _L = _INFO.num_cores, _INFO.num_subcores, _INFO.num_lanes
_NW = _NC * _NS
_CH = 8192


def _make_sc_sums(B, C, V):
    per_w = V // _NW
    n_ch = per_w // _CH
    mesh = plsc.VectorSubcoreMesh(core_axis_name="c", subcore_axis_name="s")

    @functools.partial(
        pl.kernel,
        mesh=mesh,
        out_type=jax.ShapeDtypeStruct((_NW * B * 3 * _L,), jnp.float32),
        scratch_types=[
            pltpu.VMEM((_CH,), jnp.float32),
            pltpu.VMEM((_CH,), jnp.int32),
            pltpu.VMEM((B * 3 * _L,), jnp.float32),
        ],
    )
    def sc_sums(x_hbm, t_hbm, out_hbm, xv, tv, outv):
        wid = lax.axis_index("s") * _NC + lax.axis_index("c")
        base = wid * per_w
        for b in range(B):
            S = jnp.zeros((_L,), jnp.float32)
            T = jnp.zeros((_L,), jnp.float32)
            N = jnp.zeros((_L,), jnp.float32)
            for ci in range(n_ch):
                off = base + ci * _CH
                pltpu.sync_copy(
                    x_hbm.at[pl.ds((b * C + (C - 1)) * V + off, _CH)], xv)
                pltpu.sync_copy(t_hbm.at[pl.ds(b * V + off, _CH)], tv)

                def inner(i, carry):
                    s, t, n = carry
                    xs = xv[pl.ds(i * _L, _L)]
                    ts = tv[pl.ds(i * _L, _L)]
                    m = ts == (C - 1)
                    s = s + xs
                    t = t + jnp.where(m, xs, 0.0)
                    n = n + jnp.where(m, 1.0, 0.0)
                    return s, t, n

                S, T, N = lax.fori_loop(0, _CH // _L, inner, (S, T, N),
                                        unroll=8)
            outv[pl.ds((b * 3 + 0) * _L, _L)] = S
            outv[pl.ds((b * 3 + 1) * _L, _L)] = T
            outv[pl.ds((b * 3 + 2) * _L, _L)] = N
        pltpu.sync_copy(outv, out_hbm.at[pl.ds(wid * B * 3 * _L, B * 3 * _L)])

    return sc_sums


def kernel(inputs, targets):
    B, C, D, H, W = inputs.shape
    V = D * H * W
    xf = inputs.reshape(-1)
    tf = targets.reshape(-1)
    part = _make_sc_sums(B, C, V)(xf, tf)
    sums = part.reshape(_NW, B, 3, _L).sum(axis=(0, 3))
    S, T, N = sums[:, 0], sums[:, 1], sums[:, 2]
    tversky = (T + _SMOOTH) / (T + _ALPHA * (N - T) + _BETA * (S - T) + _SMOOTH)
    return -tversky.mean()


# speedup vs baseline: 2.4741x; 1.4223x over previous
import functools

import jax
import jax.numpy as jnp
from jax import lax
from jax.experimental import pallas as pl
from jax.experimental.pallas import tpu as pltpu, tpu_sc as plsc

_ALPHA = 0.7
_BETA = 0.3
_SMOOTH = 1.0

_INFO = plsc.get_sparse_core_info()
_NC, _NS, _L = _INFO.num_cores, _INFO.num_subcores, _INFO.num_lanes
_NW = _NC * _NS
_CH = 8192


_NBUF = 4


def _make_sc_sums(B, C, V):
    per_w = V // _NW
    n_ch = per_w // _CH
    mesh = plsc.VectorSubcoreMesh(core_axis_name="c", subcore_axis_name="s")
    scratch = (
        [pltpu.VMEM((_CH,), jnp.float32) for _ in range(_NBUF)]
        + [pltpu.VMEM((_CH,), jnp.int32) for _ in range(_NBUF)]
        + [pltpu.VMEM((B * 3 * _L,), jnp.float32)]
        + [pltpu.SemaphoreType.DMA for _ in range(2 * _NBUF)]
    )

    @functools.partial(
        pl.kernel,
        mesh=mesh,
        out_type=jax.ShapeDtypeStruct((_NW * B * 3 * _L,), jnp.float32),
        scratch_types=scratch,
    )
    def sc_sums(x_hbm, t_hbm, out_hbm, *sc):
        xbufs = sc[:_NBUF]
        tbufs = sc[_NBUF:2 * _NBUF]
        outv = sc[2 * _NBUF]
        sxs = sc[2 * _NBUF + 1:2 * _NBUF + 1 + _NBUF]
        sts = sc[2 * _NBUF + 1 + _NBUF:]
        wid = lax.axis_index("s") * _NC + lax.axis_index("c")
        base = wid * per_w
        steps = [(b, ci) for b in range(B) for ci in range(n_ch)]

        def issue(i):
            b, ci = steps[i]
            off = base + ci * _CH
            k = i % _NBUF
            cx = pltpu.async_copy(
                x_hbm.at[pl.ds((b * C + (C - 1)) * V + off, _CH)],
                xbufs[k], sxs[k])
            ct = pltpu.async_copy(
                t_hbm.at[pl.ds(b * V + off, _CH)], tbufs[k], sts[k])
            return cx, ct

        pend = {}
        for i in range(min(_NBUF, len(steps))):
            pend[i] = issue(i)
        acc = {
            b: tuple(jnp.zeros((_L,), jnp.float32) for _ in range(3))
            for b in range(B)
        }
        for i in range(len(steps)):
            b, _ = steps[i]
            k = i % _NBUF
            cx, ct = pend.pop(i)
            cx.wait()
            ct.wait()
            xv, tv = xbufs[k], tbufs[k]

            def inner(j, carry, xv=xv, tv=tv):
                s, t, n = carry
                xs = xv[pl.ds(j * _L, _L)]
                ts = tv[pl.ds(j * _L, _L)]
                m = ts == (C - 1)
                s = s + xs
                t = t + jnp.where(m, xs, 0.0)
                n = n + jnp.where(m, 1.0, 0.0)
                return s, t, n

            acc[b] = lax.fori_loop(0, _CH // _L, inner, acc[b], unroll=8)
            if i + _NBUF < len(steps):
                pend[i + _NBUF] = issue(i + _NBUF)
        for b in range(B):
            S, T, N = acc[b]
            outv[pl.ds((b * 3 + 0) * _L, _L)] = S
            outv[pl.ds((b * 3 + 1) * _L, _L)] = T
            outv[pl.ds((b * 3 + 2) * _L, _L)] = N
        pltpu.sync_copy(outv, out_hbm.at[pl.ds(wid * B * 3 * _L, B * 3 * _L)])

    return sc_sums


def kernel(inputs, targets):
    B, C, D, H, W = inputs.shape
    V = D * H * W
    xf = inputs.reshape(-1)
    tf = targets.reshape(-1)
    part = _make_sc_sums(B, C, V)(xf, tf)
    sums = part.reshape(_NW, B, 3, _L).sum(axis=(0, 3))
    S, T, N = sums[:, 0], sums[:, 1], sums[:, 2]
    tversky = (T + _SMOOTH) / (T + _ALPHA * (N - T) + _BETA * (S - T) + _SMOOTH)
    return -tversky.mean()


# speedup vs baseline: 2.6600x; 1.0751x over previous
import functools

import jax
import jax.numpy as jnp
from jax import lax
from jax.experimental import pallas as pl
from jax.experimental.pallas import tpu as pltpu, tpu_sc as plsc

_ALPHA = 0.7
_BETA = 0.3
_SMOOTH = 1.0

_INFO = plsc.get_sparse_core_info()
_NC, _NS, _L = _INFO.num_cores, _INFO.num_subcores, _INFO.num_lanes
_NW = _NC * _NS
_NBUF = 4
_DSC = 32
_DBLK = 16


def _make_sc_sums(B, C, V, v_sc):
    per_w = v_sc // _NW
    ch = min(8192, per_w)
    n_ch = per_w // ch
    mesh = plsc.VectorSubcoreMesh(core_axis_name="c", subcore_axis_name="s")
    scratch = (
        [pltpu.VMEM((ch,), jnp.float32) for _ in range(_NBUF)]
        + [pltpu.VMEM((ch,), jnp.int32) for _ in range(_NBUF)]
        + [pltpu.VMEM((B * 3 * _L,), jnp.float32)]
        + [pltpu.SemaphoreType.DMA for _ in range(2 * _NBUF)]
    )

    @functools.partial(
        pl.kernel,
        mesh=mesh,
        out_type=jax.ShapeDtypeStruct((_NW * B * 3 * _L,), jnp.float32),
        scratch_types=scratch,
    )
    def sc_sums(x_hbm, t_hbm, out_hbm, *sc):
        xbufs = sc[:_NBUF]
        tbufs = sc[_NBUF:2 * _NBUF]
        outv = sc[2 * _NBUF]
        sxs = sc[2 * _NBUF + 1:2 * _NBUF + 1 + _NBUF]
        sts = sc[2 * _NBUF + 1 + _NBUF:]
        wid = lax.axis_index("s") * _NC + lax.axis_index("c")
        base = wid * per_w
        steps = [(b, ci) for b in range(B) for ci in range(n_ch)]

        def issue(i):
            b, ci = steps[i]
            off = base + ci * ch
            k = i % _NBUF
            cx = pltpu.async_copy(
                x_hbm.at[pl.ds((b * C + (C - 1)) * V + off, ch)],
                xbufs[k], sxs[k])
            ct = pltpu.async_copy(
                t_hbm.at[pl.ds(b * V + off, ch)], tbufs[k], sts[k])
            return cx, ct

        pend = {}
        for i in range(min(_NBUF, len(steps))):
            pend[i] = issue(i)
        acc = {
            b: tuple(jnp.zeros((_L,), jnp.float32) for _ in range(3))
            for b in range(B)
        }
        for i in range(len(steps)):
            b, _ = steps[i]
            k = i % _NBUF
            cx, ct = pend.pop(i)
            cx.wait()
            ct.wait()
            xv, tv = xbufs[k], tbufs[k]

            def inner(j, carry, xv=xv, tv=tv):
                s, t, n = carry
                xs = xv[pl.ds(j * _L, _L)]
                ts = tv[pl.ds(j * _L, _L)]
                m = ts == (C - 1)
                s = s + xs
                t = t + jnp.where(m, xs, 0.0)
                n = n + jnp.where(m, 1.0, 0.0)
                return s, t, n

            acc[b] = lax.fori_loop(0, ch // _L, inner, acc[b], unroll=8)
            if i + _NBUF < len(steps):
                pend[i + _NBUF] = issue(i + _NBUF)
        for b in range(B):
            S, T, N = acc[b]
            outv[pl.ds((b * 3 + 0) * _L, _L)] = S
            outv[pl.ds((b * 3 + 1) * _L, _L)] = T
            outv[pl.ds((b * 3 + 2) * _L, _L)] = N
        pltpu.sync_copy(outv, out_hbm.at[pl.ds(wid * B * 3 * _L, B * 3 * _L)])

    return sc_sums


def _tc_sums_body(x_ref, t_ref, o_ref):
    d = pl.program_id(1)
    xb = x_ref[0, 0]
    m = (t_ref[0] == 3).astype(jnp.float32)
    xr = xb.reshape(-1, 8, 128)
    mr = m.reshape(-1, 8, 128)
    part = jnp.stack([xr.sum(0), (xr * mr).sum(0), mr.sum(0)])[None]

    @pl.when(d == 0)
    def _():
        o_ref[...] = jnp.zeros_like(o_ref)

    o_ref[...] += part


def kernel(inputs, targets):
    B, C, D, H, W = inputs.shape
    V = D * H * W
    v_sc = _DSC * H * W
    xf = inputs.reshape(-1)
    tf = targets.reshape(-1)
    sc_part = _make_sc_sums(B, C, V, v_sc)(xf, tf)
    d_off = _DSC // _DBLK
    tc_part = pl.pallas_call(
        _tc_sums_body,
        grid=(B, (D - _DSC) // _DBLK),
        in_specs=[
            pl.BlockSpec((1, 1, _DBLK, H, W),
                         lambda b, d: (b, C - 1, d + d_off, 0, 0)),
            pl.BlockSpec((1, _DBLK, H, W), lambda b, d: (b, d + d_off, 0, 0)),
        ],
        out_specs=pl.BlockSpec((1, 3, 8, 128), lambda b, d: (b, 0, 0, 0)),
        out_shape=jax.ShapeDtypeStruct((B, 3, 8, 128), jnp.float32),
    )(inputs, targets)
    sums = (sc_part.reshape(_NW, B, 3, _L).sum(axis=(0, 3))
            + tc_part.sum(axis=(2, 3)))
    S, T, N = sums[:, 0], sums[:, 1], sums[:, 2]
    tversky = (T + _SMOOTH) / (T + _ALPHA * (N - T) + _BETA * (S - T) + _SMOOTH)
    return -tversky.mean()
